# baseline (speedup 1.0000x reference)
import functools

import jax
import jax.numpy as jnp
from jax import lax
from jax.experimental import pallas as pl
from jax.experimental.pallas import tpu as pltpu
from jax.experimental.pallas import tpu_sc as plsc

_NC = 2
_NS = 16
_NW = _NC * _NS


def _sc_gather(track_table, t_idx, context_table, c_idx):
    B = t_idx.shape[0]
    D = track_table.shape[1]
    bpw = B // _NW
    mesh = plsc.VectorSubcoreMesh(core_axis_name="c", subcore_axis_name="s")

    @functools.partial(
        pl.kernel,
        out_type=(
            jax.ShapeDtypeStruct((B, D), jnp.float32),
            jax.ShapeDtypeStruct((B, D), jnp.float32),
        ),
        mesh=mesh,
        compiler_params=pltpu.CompilerParams(use_tc_tiling_on_sc=True),
        scratch_types=[
            pltpu.VMEM((bpw + 16,), jnp.int32),
            pltpu.VMEM((bpw + 16,), jnp.int32),
            pltpu.VMEM((bpw // 4, D), jnp.float32),
            pltpu.VMEM((bpw // 4, D), jnp.float32),
            pltpu.SemaphoreType.DMA,
            pltpu.SemaphoreType.DMA,
        ],
    )
    def gather_kernel(tt_hbm, ti_hbm, ct_hbm, ci_hbm, out_t, out_c,
                      ti_v, ci_v, tr_v, cr_v, sem_t, sem_c):
        wid = lax.axis_index("s") * _NC + lax.axis_index("c")
        base = wid * bpw
        pltpu.sync_copy(ti_hbm.at[pl.ds(base, bpw)], ti_v.at[pl.ds(0, bpw)])
        pltpu.sync_copy(ci_hbm.at[pl.ds(base, bpw)], ci_v.at[pl.ds(0, bpw)])

        part = bpw // 4
        for p in range(4):
            def fire(c, _, p=p):
                vt = ti_v[pl.ds(p * part + c * 8, 16)]
                vc = ci_v[pl.ds(p * part + c * 8, 16)]
                for l in range(8):
                    t = vt[l]
                    lt = (lax.shift_right_logical(t, 14) << 12) + (t & 4095)
                    pltpu.async_copy(tt_hbm.at[lt], tr_v.at[c * 8 + l], sem_t)
                    u = vc[l]
                    lu = (lax.shift_right_logical(u, 14) << 12) + (u & 4095)
                    pltpu.async_copy(ct_hbm.at[lu], cr_v.at[c * 8 + l], sem_c)
                return _

            lax.fori_loop(0, part // 8, fire, 0)
            pltpu.make_async_copy(tt_hbm.at[pl.ds(0, part)], tr_v, sem_t).wait()
            pltpu.make_async_copy(ct_hbm.at[pl.ds(0, part)], cr_v, sem_c).wait()
            pltpu.sync_copy(tr_v, out_t.at[pl.ds(base + p * part, part)])
            pltpu.sync_copy(cr_v, out_c.at[pl.ds(base + p * part, part)])

    return gather_kernel(track_table, t_idx, context_table, c_idx)


def _tpose_body(src_ref, o_ref):
    x = src_ref[...]
    lo = lax.bitcast_convert_type(x[:32].astype(jnp.bfloat16),
                                  jnp.uint16).astype(jnp.uint32)
    hi = lax.bitcast_convert_type(x[32:].astype(jnp.bfloat16),
                                  jnp.uint16).astype(jnp.uint32)
    packed = lax.bitcast_convert_type(lo | (hi << 16), jnp.float32)
    q = packed.shape[1] // 4
    for k in range(4):
        o_ref[:, 32 * k:32 * (k + 1)] = packed[:, k * q:(k + 1) * q].T


def _relayout(tableT):
    D, V = tableT.shape
    BLK = 16384
    grid = (pl.cdiv(V, BLK),)
    return pl.pallas_call(
        _tpose_body,
        grid=grid,
        in_specs=[pl.BlockSpec((D, BLK), lambda i: (0, i))],
        out_specs=pl.BlockSpec((BLK // 4, 128), lambda i: (i, 0)),
        out_shape=jax.ShapeDtypeStruct((pl.cdiv(V, BLK) * (BLK // 4), 128),
                                       jnp.float32),
    )(tableT)


def _unpack(p):
    u = lax.bitcast_convert_type(p, jnp.uint32)
    lo = lax.bitcast_convert_type((u & 0xFFFF).astype(jnp.uint16), jnp.bfloat16)
    hi = lax.bitcast_convert_type((u >> 16).astype(jnp.uint16), jnp.bfloat16)
    return jnp.concatenate([lo, hi], axis=1).astype(jnp.float32)


def _group(rows, v):
    g = lax.shift_right_logical(v, 12) & 3
    x = jnp.where(g == 0, rows[:, 0:32], 0.0)
    for k in range(1, 4):
        x = x + jnp.where(g == k, rows[:, 32 * k:32 * k + 32], 0.0)
    return _unpack(x)


def _mlp_body(te_ref, ce_ref, ti_ref, ci_ref, w1a_ref, w1b_ref, b1_ref,
              w2_ref, b2_ref, w3_ref, b3_ref, o_ref):
    x = jnp.dot(_group(te_ref[...], ti_ref[...]), w1a_ref[...],
                preferred_element_type=jnp.float32)
    x = x + jnp.dot(_group(ce_ref[...], ci_ref[...]), w1b_ref[...],
                    preferred_element_type=jnp.float32)
    x = jnp.maximum(x + b1_ref[...], 0.0)
    x = jnp.dot(x, w2_ref[...], preferred_element_type=jnp.float32)
    x = jnp.maximum(x + b2_ref[...], 0.0)
    y = jnp.sum(x * w3_ref[...], axis=1, keepdims=True) + b3_ref[...]
    o_ref[...] = jax.nn.sigmoid(y)


def _mlp(te, ce, ti, ci, W1a, W1b, b1, W2, b2, w3row, b3):
    B, L = te.shape
    D = W1a.shape[0]
    H1 = W1a.shape[1]
    H2 = W2.shape[1]
    BB = 2048
    grid = (B // BB,)
    return pl.pallas_call(
        _mlp_body,
        grid=grid,
        in_specs=[
            pl.BlockSpec((BB, L), lambda i: (i, 0)),
            pl.BlockSpec((BB, L), lambda i: (i, 0)),
            pl.BlockSpec((BB, 1), lambda i: (i, 0)),
            pl.BlockSpec((BB, 1), lambda i: (i, 0)),
            pl.BlockSpec((D, H1), lambda i: (0, 0)),
            pl.BlockSpec((D, H1), lambda i: (0, 0)),
            pl.BlockSpec((1, H1), lambda i: (0, 0)),
            pl.BlockSpec((H1, H2), lambda i: (0, 0)),
            pl.BlockSpec((1, H2), lambda i: (0, 0)),
            pl.BlockSpec((1, H2), lambda i: (0, 0)),
            pl.BlockSpec((1, 1), lambda i: (0, 0)),
        ],
        out_specs=pl.BlockSpec((BB, 1), lambda i: (i, 0)),
        out_shape=jax.ShapeDtypeStruct((B, 1), jnp.float32),
    )(te, ce, ti, ci, W1a, W1b, b1, W2, b2, w3row, b3)


def kernel(track_indices, context_indices, track_table, context_table,
           W1, b1, W2, b2, W3, b3):
    ti = track_indices.astype(jnp.int32)
    ci = context_indices.astype(jnp.int32)
    tt_rm = _relayout(track_table.T)
    ct_rm = _relayout(context_table.T)
    te, ce = _sc_gather(tt_rm, ti, ct_rm, ci)
    B = ti.shape[0]
    D = track_table.shape[1]
    W1a, W1b = W1[:D], W1[D:]
    return _mlp(te, ce, ti.reshape(B, 1), ci.reshape(B, 1),
                W1a, W1b, b1.reshape(1, -1), W2, b2.reshape(1, -1),
                W3.reshape(1, -1), b3.reshape(1, 1))


# speedup vs baseline: 1.0586x; 1.0586x over previous
import functools

import jax
import jax.numpy as jnp
from jax import lax
from jax.experimental import pallas as pl
from jax.experimental.pallas import tpu as pltpu
from jax.experimental.pallas import tpu_sc as plsc

_NC = 2
_NS = 16
_NW = _NC * _NS


def _sc_gather(track_table, t_idx, context_table, c_idx):
    B = t_idx.shape[0]
    D = track_table.shape[1]
    bpw = B // _NW
    mesh = plsc.VectorSubcoreMesh(core_axis_name="c", subcore_axis_name="s")

    @functools.partial(
        pl.kernel,
        out_type=(
            jax.ShapeDtypeStruct((B, D), jnp.float32),
            jax.ShapeDtypeStruct((B, D), jnp.float32),
        ),
        mesh=mesh,
        compiler_params=pltpu.CompilerParams(use_tc_tiling_on_sc=True),
        scratch_types=[
            pltpu.VMEM((bpw + 16,), jnp.int32),
            pltpu.VMEM((bpw + 16,), jnp.int32),
            pltpu.VMEM((bpw // 2, D), jnp.float32),
            pltpu.VMEM((bpw // 2, D), jnp.float32),
            pltpu.SemaphoreType.DMA,
            pltpu.SemaphoreType.DMA,
        ],
    )
    def gather_kernel(tt_hbm, ti_hbm, ct_hbm, ci_hbm, out_t, out_c,
                      ti_v, ci_v, tr_v, cr_v, sem_t, sem_c):
        wid = lax.axis_index("s") * _NC + lax.axis_index("c")
        base = wid * bpw
        pltpu.sync_copy(ti_hbm.at[pl.ds(base, bpw)], ti_v.at[pl.ds(0, bpw)])
        pltpu.sync_copy(ci_hbm.at[pl.ds(base, bpw)], ci_v.at[pl.ds(0, bpw)])

        half = bpw // 2
        for p in range(2):
            def fire(c, _, p=p):
                vt = ti_v[pl.ds(p * half + c * 8, 16)]
                vc = ci_v[pl.ds(p * half + c * 8, 16)]
                for l in range(8):
                    pltpu.async_copy(tt_hbm.at[vt[l]], tr_v.at[c * 8 + l], sem_t)
                    pltpu.async_copy(ct_hbm.at[vc[l]], cr_v.at[c * 8 + l], sem_c)
                return _

            lax.fori_loop(0, half // 8, fire, 0)
            pltpu.make_async_copy(tt_hbm.at[pl.ds(0, half)], tr_v, sem_t).wait()
            pltpu.make_async_copy(ct_hbm.at[pl.ds(0, half)], cr_v, sem_c).wait()
            pltpu.sync_copy(tr_v, out_t.at[pl.ds(base + p * half, half)])
            pltpu.sync_copy(cr_v, out_c.at[pl.ds(base + p * half, half)])

    return gather_kernel(track_table, t_idx, context_table, c_idx)


def _tpose_body(src_ref, o_ref):
    o_ref[...] = src_ref[...].T


def _relayout(tableT):
    D, V = tableT.shape
    BLK = 32768
    grid = (pl.cdiv(V, BLK),)
    return pl.pallas_call(
        _tpose_body,
        grid=grid,
        in_specs=[pl.BlockSpec((D, BLK), lambda i: (0, i))],
        out_specs=pl.BlockSpec((BLK, D), lambda i: (i, 0)),
        out_shape=jax.ShapeDtypeStruct((V, D), jnp.float32),
    )(tableT)


def _mlp_body(te_ref, ce_ref, w1a_ref, w1b_ref, b1_ref, w2_ref, b2_ref,
              w3_ref, b3_ref, o_ref):
    x = jnp.dot(te_ref[...], w1a_ref[...], preferred_element_type=jnp.float32)
    x = x + jnp.dot(ce_ref[...], w1b_ref[...], preferred_element_type=jnp.float32)
    x = jnp.maximum(x + b1_ref[...], 0.0)
    x = jnp.dot(x, w2_ref[...], preferred_element_type=jnp.float32)
    x = jnp.maximum(x + b2_ref[...], 0.0)
    y = jnp.sum(x * w3_ref[...], axis=1, keepdims=True) + b3_ref[...]
    o_ref[...] = jax.nn.sigmoid(y)


def _mlp(te, ce, W1a, W1b, b1, W2, b2, w3row, b3):
    B, D = te.shape
    H1 = W1a.shape[1]
    H2 = W2.shape[1]
    BB = 2048
    grid = (B // BB,)
    return pl.pallas_call(
        _mlp_body,
        grid=grid,
        in_specs=[
            pl.BlockSpec((BB, D), lambda i: (i, 0)),
            pl.BlockSpec((BB, D), lambda i: (i, 0)),
            pl.BlockSpec((D, H1), lambda i: (0, 0)),
            pl.BlockSpec((D, H1), lambda i: (0, 0)),
            pl.BlockSpec((1, H1), lambda i: (0, 0)),
            pl.BlockSpec((H1, H2), lambda i: (0, 0)),
            pl.BlockSpec((1, H2), lambda i: (0, 0)),
            pl.BlockSpec((1, H2), lambda i: (0, 0)),
            pl.BlockSpec((1, 1), lambda i: (0, 0)),
        ],
        out_specs=pl.BlockSpec((BB, 1), lambda i: (i, 0)),
        out_shape=jax.ShapeDtypeStruct((B, 1), jnp.float32),
    )(te, ce, W1a, W1b, b1, W2, b2, w3row, b3)


def kernel(track_indices, context_indices, track_table, context_table,
           W1, b1, W2, b2, W3, b3):
    ti = track_indices.astype(jnp.int32)
    ci = context_indices.astype(jnp.int32)
    tt_rm = _relayout(track_table.T)
    ct_rm = _relayout(context_table.T)
    te, ce = _sc_gather(tt_rm, ti, ct_rm, ci)
    D = track_table.shape[1]
    W1a, W1b = W1[:D], W1[D:]
    return _mlp(te, ce, W1a, W1b, b1.reshape(1, -1), W2, b2.reshape(1, -1),
                W3.reshape(1, -1), b3.reshape(1, 1))
